# Initial kernel scaffold; baseline (speedup 1.0000x reference)
import jax
import jax.numpy as jnp
from jax.experimental import pallas as pl


def kernel(p1, p2):
    raise NotImplementedError("write your pallas kernel here")


# speedup vs baseline: 73.3237x; 73.3237x over previous
import jax
import jax.numpy as jnp
from jax import lax
from jax.experimental import pallas as pl

_N = 2048
_BLK = 256
_NBLK = _N // _BLK
_STEPLIM = 6
_TOL = 1e-06


def _knn_pass(aT, b_rows, bT8):
    ax = aT[0:1, :]
    ay = aT[1:2, :]
    az = aT[2:3, :]
    run_min = jnp.full((1, _N), 3.0e38, jnp.float32)
    run_idx = jnp.zeros((1, _N), jnp.int32)
    for blk in range(_NBLK):
        j0 = blk * _BLK
        bx = b_rows[j0:j0 + _BLK, 0:1]
        by = b_rows[j0:j0 + _BLK, 1:2]
        bz = b_rows[j0:j0 + _BLK, 2:3]
        dx = ax - bx
        dy = ay - by
        dz = az - bz
        d2 = dx * dx + dy * dy + dz * dz
        bmin = jnp.min(d2, axis=0, keepdims=True)
        jj = lax.broadcasted_iota(jnp.int32, (_BLK, _N), 0) + j0
        bidx = jnp.min(jnp.where(d2 == bmin, jj, _N), axis=0, keepdims=True)
        upd = bmin < run_min
        run_idx = jnp.where(upd, bidx, run_idx)
        run_min = jnp.where(upd, bmin, run_min)
    M = jnp.zeros((8, _N), jnp.float32)
    for blk in range(_NBLK):
        j0 = blk * _BLK
        jj = lax.broadcasted_iota(jnp.int32, (_BLK, _N), 0) + j0
        oh = (jj == run_idx).astype(jnp.float32)
        M = M + jnp.dot(bT8[:, j0:j0 + _BLK], oh,
                        preferred_element_type=jnp.float32)
    err = jnp.sqrt(run_min)
    return err, M


def _kabsch(pT, mT):
    inv_n = jnp.float32(1.0 / _N)
    mu1 = [jnp.sum(pT[k:k + 1, :], axis=1, keepdims=True) * inv_n
           for k in range(3)]
    mu2 = [jnp.sum(mT[k:k + 1, :], axis=1, keepdims=True) * inv_n
           for k in range(3)]
    H = [[jnp.sum((pT[i:i + 1, :] - mu1[i]) * (mT[j:j + 1, :] - mu2[j]),
                  axis=1, keepdims=True)
          for j in range(3)] for i in range(3)]

    one = jnp.ones((1, 1), jnp.float32)
    zero = jnp.zeros((1, 1), jnp.float32)
    K = [[H[0][i] * H[0][j] + H[1][i] * H[1][j] + H[2][i] * H[2][j]
          for j in range(3)] for i in range(3)]
    V = [[one if i == j else zero for j in range(3)] for i in range(3)]

    for _ in range(6):
        for (p, q) in ((0, 1), (0, 2), (1, 2)):
            apq = K[p][q]
            small = jnp.abs(apq) < 1e-30
            apq_safe = jnp.where(small, one, apq)
            tau = (K[q][q] - K[p][p]) * 0.5 / apq_safe
            sgn = jnp.where(tau >= 0.0, 1.0, -1.0)
            tt = sgn / (jnp.abs(tau) + jnp.sqrt(1.0 + tau * tau))
            t_ = jnp.where(small, zero, tt)
            c = 1.0 / jnp.sqrt(1.0 + t_ * t_)
            s = t_ * c
            Kn = [row[:] for row in K]
            kpp = K[p][p]
            kqq = K[q][q]
            Kn[p][p] = c * c * kpp - 2.0 * s * c * apq + s * s * kqq
            Kn[q][q] = s * s * kpp + 2.0 * s * c * apq + c * c * kqq
            Kn[p][q] = zero
            Kn[q][p] = zero
            r = 3 - p - q
            krp = K[r][p]
            krq = K[r][q]
            Kn[r][p] = c * krp - s * krq
            Kn[p][r] = Kn[r][p]
            Kn[r][q] = s * krp + c * krq
            Kn[q][r] = Kn[r][q]
            K = Kn
            Vn = [row[:] for row in V]
            for rr in range(3):
                vrp = V[rr][p]
                vrq = V[rr][q]
                Vn[rr][p] = c * vrp - s * vrq
                Vn[rr][q] = s * vrp + c * vrq
            V = Vn

    e = [K[0][0], K[1][1], K[2][2]]

    def cswap(e, V, i, j):
        sw = e[i] < e[j]
        e2 = e[:]
        e2[i] = jnp.where(sw, e[j], e[i])
        e2[j] = jnp.where(sw, e[i], e[j])
        V2 = [row[:] for row in V]
        for r in range(3):
            V2[r][i] = jnp.where(sw, V[r][j], V[r][i])
            V2[r][j] = jnp.where(sw, V[r][i], V[r][j])
        return e2, V2

    e, V = cswap(e, V, 0, 1)
    e, V = cswap(e, V, 1, 2)
    e, V = cswap(e, V, 0, 1)

    sv = [jnp.sqrt(jnp.maximum(e[i], 0.0)) for i in range(3)]
    svs = [jnp.maximum(sv[i], 1e-30) for i in range(3)]
    U = [[(H[m][0] * V[0][i] + H[m][1] * V[1][i] + H[m][2] * V[2][i]) / svs[i]
          for i in range(3)] for m in range(3)]
    detH = (H[0][0] * (H[1][1] * H[2][2] - H[1][2] * H[2][1])
            - H[0][1] * (H[1][0] * H[2][2] - H[1][2] * H[2][0])
            + H[0][2] * (H[1][0] * H[2][1] - H[1][1] * H[2][0]))
    d = jnp.where(detH >= 0.0, 1.0, -1.0)
    dd = [one, one, d]
    R = [[dd[0] * V[a][0] * U[b][0] + dd[1] * V[a][1] * U[b][1]
          + dd[2] * V[a][2] * U[b][2]
          for b in range(3)] for a in range(3)]
    t = [mu2[a] - (R[a][0] * mu1[0] + R[a][1] * mu1[1] + R[a][2] * mu1[2])
         for a in range(3)]
    return R, t


def _apply(R, t, aT):
    rows = [R[k][0] * aT[0:1, :] + R[k][1] * aT[1:2, :]
            + R[k][2] * aT[2:3, :] + t[k] for k in range(3)]
    return jnp.concatenate(rows, axis=0)


def _icp_body(aT_ref, b_ref, bT8_ref, out_ref):
    aT0 = aT_ref[...]
    b_rows = b_ref[...]
    bT8 = bT8_ref[...]

    err0, M0 = _knn_pass(aT0, b_rows, bT8)
    R0, t0 = _kabsch(aT0, M0[0:3, :])
    aT1 = _apply(R0, t0, aT0)

    def cond_fun(state):
        it, aT, err, conv = state
        return jnp.logical_and(it <= _STEPLIM, jnp.logical_not(conv))

    def body_fun(state):
        it, aT, err, conv = state
        errnew, M = _knn_pass(aT, b_rows, bT8)
        R, t = _kabsch(aT, M[0:3, :])
        aTn = _apply(R, t, aT)
        convn = jnp.all(jnp.abs((errnew - err) / err) < _TOL)
        errout = jnp.where(convn, err, errnew)
        return (it + 1, aTn, errout, convn)

    state = (jnp.asarray(1, jnp.int32), aT1, err0, jnp.asarray(False))
    _, aTf, _, _ = lax.while_loop(cond_fun, body_fun, state)

    Rf, tf = _kabsch(aT0, aTf)
    rows = [jnp.concatenate([Rf[k][0], Rf[k][1], Rf[k][2], tf[k]], axis=1)
            for k in range(3)]
    out_ref[...] = jnp.concatenate(rows, axis=0)


def kernel(p1, p2):
    a = p1[0]
    b = p2[0]
    aT = a.T
    bT8 = jnp.concatenate([b.T, jnp.zeros((5, _N), jnp.float32)], axis=0)
    out = pl.pallas_call(
        _icp_body,
        out_shape=jax.ShapeDtypeStruct((3, 4), jnp.float32),
    )(aT, b, bT8)
    return out[None]
